# baseline (speedup 1.0000x reference)
import jax
import jax.numpy as jnp
from jax.experimental import pallas as pl
from jax.experimental.pallas import tpu as pltpu

_TS = 512
_TF = 256


def _fused_swiglu(x_ref, wup_ref, wg_ref, wd_ref, out_ref):
    f = pl.program_id(1)
    x = x_ref[...]
    up = jax.lax.dot_general(
        x, wup_ref[...], (((1,), (1,)), ((), ())),
        preferred_element_type=jnp.float32)
    gate = jnp.dot(x, wg_ref[...], preferred_element_type=jnp.float32)
    z = up * gate * jax.lax.logistic(gate)
    acc = jnp.dot(z, wd_ref[...], preferred_element_type=jnp.float32)

    @pl.when(f == 0)
    def _():
        out_ref[...] = acc

    @pl.when(f > 0)
    def _():
        out_ref[...] = out_ref[...] + acc


def kernel(x, Wup, Wgatet, Wdownt):
    B, S, D = x.shape
    F = Wup.shape[0]
    x2 = x.reshape(S, D)
    out = pl.pallas_call(
        _fused_swiglu,
        grid=(S // _TS, F // _TF),
        in_specs=[
            pl.BlockSpec((_TS, D), lambda s, f: (s, 0)),
            pl.BlockSpec((_TF, D), lambda s, f: (f, 0)),
            pl.BlockSpec((D, _TF), lambda s, f: (0, f)),
            pl.BlockSpec((_TF, D), lambda s, f: (f, 0)),
        ],
        out_specs=pl.BlockSpec((_TS, D), lambda s, f: (s, 0)),
        out_shape=jax.ShapeDtypeStruct((S, D), jnp.float32),
        compiler_params=pltpu.CompilerParams(
            dimension_semantics=("parallel", "arbitrary")),
    )(x2, Wup, Wgatet, Wdownt)
    return out.reshape(B, S, D)


# speedup vs baseline: 1.2600x; 1.2600x over previous
import jax
import jax.numpy as jnp
from jax.experimental import pallas as pl
from jax.experimental.pallas import tpu as pltpu

_TF = 256


def _fused_swiglu(x_ref, wup_ref, wg_ref, wd_ref, out_ref):
    f = pl.program_id(0)
    x = x_ref[...]
    wup = wup_ref[...].astype(jnp.bfloat16)
    wg = wg_ref[...].astype(jnp.bfloat16)
    wd = wd_ref[...].astype(jnp.bfloat16)
    up = jax.lax.dot_general(
        x, wup, (((1,), (1,)), ((), ())),
        preferred_element_type=jnp.float32)
    gate = jnp.dot(x, wg, preferred_element_type=jnp.float32)
    z = (up * gate * jax.lax.logistic(gate)).astype(jnp.bfloat16)
    acc = jnp.dot(z, wd, preferred_element_type=jnp.float32)

    @pl.when(f == 0)
    def _():
        out_ref[...] = acc

    @pl.when(f > 0)
    def _():
        out_ref[...] = out_ref[...] + acc


def kernel(x, Wup, Wgatet, Wdownt):
    B, S, D = x.shape
    F = Wup.shape[0]
    xb = x.reshape(S, D).astype(jnp.bfloat16)
    out = pl.pallas_call(
        _fused_swiglu,
        grid=(F // _TF,),
        in_specs=[
            pl.BlockSpec((S, D), lambda f: (0, 0)),
            pl.BlockSpec((_TF, D), lambda f: (f, 0)),
            pl.BlockSpec((D, _TF), lambda f: (0, f)),
            pl.BlockSpec((_TF, D), lambda f: (f, 0)),
        ],
        out_specs=pl.BlockSpec((S, D), lambda f: (0, 0)),
        out_shape=jax.ShapeDtypeStruct((S, D), jnp.float32),
        compiler_params=pltpu.CompilerParams(
            dimension_semantics=("arbitrary",)),
    )(xb, Wup, Wgatet, Wdownt)
    return out.reshape(B, S, D)
